# initial kernel scaffold (unmeasured)
import jax
import jax.numpy as jnp
from jax import lax
from jax.experimental import pallas as pl
from jax.experimental.pallas import tpu as pltpu


def kernel(
    x,
):
    def body(*refs):
        pass

    out_shape = jax.ShapeDtypeStruct(..., jnp.float32)
    return pl.pallas_call(body, out_shape=out_shape)(...)


# baseline (device time: 83433 ns/iter reference)
import jax
import jax.numpy as jnp
from jax import lax
from jax.experimental import pallas as pl
from jax.experimental.pallas import tpu as pltpu


def kernel(x):
    _, m, n = x.shape
    half = m // 2

    def body(x_ref, out_ref, send_buf, rs_recv, tiles, send_sems, recv_sems):
        my_x = lax.axis_index("x")
        my_y = lax.axis_index("y")
        ox = 1 - my_x
        oy = 1 - my_y
        xp = (ox, my_y)
        yp = (my_x, oy)

        barrier = pltpu.get_barrier_semaphore()
        for nbr in (xp, yp):
            pl.semaphore_signal(
                barrier, inc=1, device_id=nbr,
                device_id_type=pl.DeviceIdType.MESH,
            )
        pl.semaphore_wait(barrier, 2)

        send_buf[...] = x_ref[0, pl.ds(ox * half, half), :].astype(jnp.bfloat16)
        rs = pltpu.make_async_remote_copy(
            src_ref=send_buf,
            dst_ref=rs_recv,
            send_sem=send_sems.at[0],
            recv_sem=recv_sems.at[0],
            device_id=xp,
            device_id_type=pl.DeviceIdType.MESH,
        )
        rs.start()
        rs.wait()

        tiles[0, :, :] = (
            x_ref[0, pl.ds(my_x * half, half), :].astype(jnp.bfloat16)
            + rs_recv[...]
        )

        ag_y = pltpu.make_async_remote_copy(
            src_ref=tiles.at[0],
            dst_ref=tiles.at[1],
            send_sem=send_sems.at[1],
            recv_sem=recv_sems.at[1],
            device_id=yp,
            device_id_type=pl.DeviceIdType.MESH,
        )
        ag_x = pltpu.make_async_remote_copy(
            src_ref=tiles.at[0],
            dst_ref=tiles.at[2],
            send_sem=send_sems.at[2],
            recv_sem=recv_sems.at[2],
            device_id=xp,
            device_id_type=pl.DeviceIdType.MESH,
        )
        ag_y.start()
        ag_x.start()

        ag_x.wait_recv()
        fwd = pltpu.make_async_remote_copy(
            src_ref=tiles.at[2],
            dst_ref=tiles.at[3],
            send_sem=send_sems.at[3],
            recv_sem=recv_sems.at[3],
            device_id=yp,
            device_id_type=pl.DeviceIdType.MESH,
        )
        fwd.start()

        ag_y.wait()
        ag_x.wait_send()
        fwd.wait()

        r_me = my_x * half
        r_ot = ox * half
        c_me = my_y * n
        c_ot = oy * n
        out_ref[pl.ds(r_me, half), pl.ds(c_me, n)] = tiles[0, :, :]
        out_ref[pl.ds(r_me, half), pl.ds(c_ot, n)] = tiles[1, :, :]
        out_ref[pl.ds(r_ot, half), pl.ds(c_me, n)] = tiles[2, :, :]
        out_ref[pl.ds(r_ot, half), pl.ds(c_ot, n)] = tiles[3, :, :]

    return pl.pallas_call(
        body,
        out_shape=jax.ShapeDtypeStruct((m, 2 * n), jnp.bfloat16),
        in_specs=[pl.BlockSpec(memory_space=pltpu.VMEM)],
        out_specs=pl.BlockSpec(memory_space=pltpu.VMEM),
        scratch_shapes=[
            pltpu.VMEM((half, n), jnp.bfloat16),
            pltpu.VMEM((half, n), jnp.bfloat16),
            pltpu.VMEM((4, half, n), jnp.bfloat16),
            pltpu.SemaphoreType.DMA((4,)),
            pltpu.SemaphoreType.DMA((4,)),
        ],
        compiler_params=pltpu.CompilerParams(collective_id=0),
    )(x)


# device time: 63701 ns/iter; 1.3098x vs baseline; 1.3098x over previous
import jax
import jax.numpy as jnp
from jax import lax
from jax.experimental import pallas as pl
from jax.experimental.pallas import tpu as pltpu

C = 4


def kernel(x):
    _, m, n = x.shape
    half = m // 2
    ck = half // C

    def body(x_ref, out_ref, send_buf, rs_recv,
             rs_ss, rs_rs, agy_ss, agy_rs, agx_ss, agx_rs, fwd_ss, fwd_rs):
        my_x = lax.axis_index("x")
        my_y = lax.axis_index("y")
        ox = 1 - my_x
        oy = 1 - my_y
        xp = (ox, my_y)
        yp = (my_x, oy)

        r_me = my_x * half
        r_ot = ox * half
        c_me = my_y * n

        barrier = pltpu.get_barrier_semaphore()
        for nbr in (xp, yp):
            pl.semaphore_signal(
                barrier, inc=1, device_id=nbr,
                device_id_type=pl.DeviceIdType.MESH,
            )
        pl.semaphore_wait(barrier, 2)

        rs = []
        for k in range(C):
            send_buf[pl.ds(k * ck, ck), :] = x_ref[
                0, pl.ds(r_ot + k * ck, ck), :
            ].astype(jnp.bfloat16)
            d = pltpu.make_async_remote_copy(
                src_ref=send_buf.at[pl.ds(k * ck, ck), :],
                dst_ref=rs_recv.at[pl.ds(k * ck, ck), :],
                send_sem=rs_ss.at[k],
                recv_sem=rs_rs.at[k],
                device_id=xp,
                device_id_type=pl.DeviceIdType.MESH,
            )
            d.start()
            rs.append(d)

        ag_y, ag_x = [], []
        for k in range(C):
            rs[k].wait_recv()
            out_ref[pl.ds(r_me + k * ck, ck), pl.ds(c_me, n)] = (
                x_ref[0, pl.ds(r_me + k * ck, ck), :].astype(jnp.bfloat16)
                + rs_recv[pl.ds(k * ck, ck), :]
            )
            src = out_ref.at[pl.ds(r_me + k * ck, ck), pl.ds(c_me, n)]
            dx = pltpu.make_async_remote_copy(
                src_ref=src, dst_ref=src,
                send_sem=agx_ss.at[k], recv_sem=agx_rs.at[k],
                device_id=xp, device_id_type=pl.DeviceIdType.MESH,
            )
            dy = pltpu.make_async_remote_copy(
                src_ref=src, dst_ref=src,
                send_sem=agy_ss.at[k], recv_sem=agy_rs.at[k],
                device_id=yp, device_id_type=pl.DeviceIdType.MESH,
            )
            dx.start()
            dy.start()
            ag_x.append(dx)
            ag_y.append(dy)

        fwd = []
        for k in range(C):
            ag_x[k].wait_recv()
            src = out_ref.at[pl.ds(r_ot + k * ck, ck), pl.ds(c_me, n)]
            d = pltpu.make_async_remote_copy(
                src_ref=src, dst_ref=src,
                send_sem=fwd_ss.at[k], recv_sem=fwd_rs.at[k],
                device_id=yp, device_id_type=pl.DeviceIdType.MESH,
            )
            d.start()
            fwd.append(d)

        for k in range(C):
            rs[k].wait_send()
            ag_x[k].wait_send()
            ag_y[k].wait()
            fwd[k].wait()

    return pl.pallas_call(
        body,
        out_shape=jax.ShapeDtypeStruct((m, 2 * n), jnp.bfloat16),
        in_specs=[pl.BlockSpec(memory_space=pltpu.VMEM)],
        out_specs=pl.BlockSpec(memory_space=pltpu.VMEM),
        scratch_shapes=[
            pltpu.VMEM((half, n), jnp.bfloat16),
            pltpu.VMEM((half, n), jnp.bfloat16),
            pltpu.SemaphoreType.DMA((C,)),
            pltpu.SemaphoreType.DMA((C,)),
            pltpu.SemaphoreType.DMA((C,)),
            pltpu.SemaphoreType.DMA((C,)),
            pltpu.SemaphoreType.DMA((C,)),
            pltpu.SemaphoreType.DMA((C,)),
            pltpu.SemaphoreType.DMA((C,)),
            pltpu.SemaphoreType.DMA((C,)),
        ],
        compiler_params=pltpu.CompilerParams(collective_id=0),
    )(x)


# device time: 61107 ns/iter; 1.3654x vs baseline; 1.0425x over previous
import jax
import jax.numpy as jnp
from jax import lax
from jax.experimental import pallas as pl
from jax.experimental.pallas import tpu as pltpu

C = 8


def kernel(x):
    _, m, n = x.shape
    half = m // 2
    ck = half // C

    def body(x_ref, out_ref, send_buf, rs_recv,
             rs_ss, rs_rs, agy_ss, agy_rs, agx_ss, agx_rs, fwd_ss, fwd_rs):
        my_x = lax.axis_index("x")
        my_y = lax.axis_index("y")
        ox = 1 - my_x
        oy = 1 - my_y
        xp = (ox, my_y)
        yp = (my_x, oy)

        r_me = my_x * half
        r_ot = ox * half
        c_me = my_y * n

        barrier = pltpu.get_barrier_semaphore()
        for nbr in (xp, yp):
            pl.semaphore_signal(
                barrier, inc=1, device_id=nbr,
                device_id_type=pl.DeviceIdType.MESH,
            )
        pl.semaphore_wait(barrier, 2)

        rs = []
        for k in range(C):
            send_buf[pl.ds(k * ck, ck), :] = x_ref[
                0, pl.ds(r_ot + k * ck, ck), :
            ].astype(jnp.bfloat16)
            d = pltpu.make_async_remote_copy(
                src_ref=send_buf.at[pl.ds(k * ck, ck), :],
                dst_ref=rs_recv.at[pl.ds(k * ck, ck), :],
                send_sem=rs_ss.at[k],
                recv_sem=rs_rs.at[k],
                device_id=xp,
                device_id_type=pl.DeviceIdType.MESH,
            )
            d.start()
            rs.append(d)

        ag_y, ag_x = [], []
        for k in range(C):
            rs[k].wait_recv()
            out_ref[pl.ds(r_me + k * ck, ck), pl.ds(c_me, n)] = (
                x_ref[0, pl.ds(r_me + k * ck, ck), :].astype(jnp.bfloat16)
                + rs_recv[pl.ds(k * ck, ck), :]
            )
            src = out_ref.at[pl.ds(r_me + k * ck, ck), pl.ds(c_me, n)]
            dx = pltpu.make_async_remote_copy(
                src_ref=src, dst_ref=src,
                send_sem=agx_ss.at[k], recv_sem=agx_rs.at[k],
                device_id=xp, device_id_type=pl.DeviceIdType.MESH,
            )
            dy = pltpu.make_async_remote_copy(
                src_ref=src, dst_ref=src,
                send_sem=agy_ss.at[k], recv_sem=agy_rs.at[k],
                device_id=yp, device_id_type=pl.DeviceIdType.MESH,
            )
            dx.start()
            dy.start()
            ag_x.append(dx)
            ag_y.append(dy)

        fwd = []
        for k in range(C):
            ag_x[k].wait_recv()
            src = out_ref.at[pl.ds(r_ot + k * ck, ck), pl.ds(c_me, n)]
            d = pltpu.make_async_remote_copy(
                src_ref=src, dst_ref=src,
                send_sem=fwd_ss.at[k], recv_sem=fwd_rs.at[k],
                device_id=yp, device_id_type=pl.DeviceIdType.MESH,
            )
            d.start()
            fwd.append(d)

        for k in range(C):
            rs[k].wait_send()
            ag_x[k].wait_send()
            ag_y[k].wait()
            fwd[k].wait()

    return pl.pallas_call(
        body,
        out_shape=jax.ShapeDtypeStruct((m, 2 * n), jnp.bfloat16),
        in_specs=[pl.BlockSpec(memory_space=pltpu.VMEM)],
        out_specs=pl.BlockSpec(memory_space=pltpu.VMEM),
        scratch_shapes=[
            pltpu.VMEM((half, n), jnp.bfloat16),
            pltpu.VMEM((half, n), jnp.bfloat16),
            pltpu.SemaphoreType.DMA((C,)),
            pltpu.SemaphoreType.DMA((C,)),
            pltpu.SemaphoreType.DMA((C,)),
            pltpu.SemaphoreType.DMA((C,)),
            pltpu.SemaphoreType.DMA((C,)),
            pltpu.SemaphoreType.DMA((C,)),
            pltpu.SemaphoreType.DMA((C,)),
            pltpu.SemaphoreType.DMA((C,)),
        ],
        compiler_params=pltpu.CompilerParams(collective_id=0),
    )(x)


# device time: 60259 ns/iter; 1.3846x vs baseline; 1.0141x over previous
import jax
import jax.numpy as jnp
from jax import lax
from jax.experimental import pallas as pl
from jax.experimental.pallas import tpu as pltpu

C = 16


def kernel(x):
    _, m, n = x.shape
    half = m // 2
    ck = half // C

    def body(x_ref, out_ref, send_buf, rs_recv,
             rs_ss, rs_rs, agy_ss, agy_rs, agx_ss, agx_rs, fwd_ss, fwd_rs):
        my_x = lax.axis_index("x")
        my_y = lax.axis_index("y")
        ox = 1 - my_x
        oy = 1 - my_y
        xp = (ox, my_y)
        yp = (my_x, oy)

        r_me = my_x * half
        r_ot = ox * half
        c_me = my_y * n

        barrier = pltpu.get_barrier_semaphore()
        for nbr in (xp, yp):
            pl.semaphore_signal(
                barrier, inc=1, device_id=nbr,
                device_id_type=pl.DeviceIdType.MESH,
            )
        pl.semaphore_wait(barrier, 2)

        rs = []
        for k in range(C):
            send_buf[pl.ds(k * ck, ck), :] = x_ref[
                0, pl.ds(r_ot + k * ck, ck), :
            ].astype(jnp.bfloat16)
            d = pltpu.make_async_remote_copy(
                src_ref=send_buf.at[pl.ds(k * ck, ck), :],
                dst_ref=rs_recv.at[pl.ds(k * ck, ck), :],
                send_sem=rs_ss.at[k],
                recv_sem=rs_rs.at[k],
                device_id=xp,
                device_id_type=pl.DeviceIdType.MESH,
            )
            d.start()
            rs.append(d)

        ag_y, ag_x = [], []
        for k in range(C):
            rs[k].wait_recv()
            out_ref[pl.ds(r_me + k * ck, ck), pl.ds(c_me, n)] = (
                x_ref[0, pl.ds(r_me + k * ck, ck), :].astype(jnp.bfloat16)
                + rs_recv[pl.ds(k * ck, ck), :]
            )
            src = out_ref.at[pl.ds(r_me + k * ck, ck), pl.ds(c_me, n)]
            dx = pltpu.make_async_remote_copy(
                src_ref=src, dst_ref=src,
                send_sem=agx_ss.at[k], recv_sem=agx_rs.at[k],
                device_id=xp, device_id_type=pl.DeviceIdType.MESH,
            )
            dy = pltpu.make_async_remote_copy(
                src_ref=src, dst_ref=src,
                send_sem=agy_ss.at[k], recv_sem=agy_rs.at[k],
                device_id=yp, device_id_type=pl.DeviceIdType.MESH,
            )
            dx.start()
            dy.start()
            ag_x.append(dx)
            ag_y.append(dy)

        fwd = []
        for k in range(C):
            ag_x[k].wait_recv()
            src = out_ref.at[pl.ds(r_ot + k * ck, ck), pl.ds(c_me, n)]
            d = pltpu.make_async_remote_copy(
                src_ref=src, dst_ref=src,
                send_sem=fwd_ss.at[k], recv_sem=fwd_rs.at[k],
                device_id=yp, device_id_type=pl.DeviceIdType.MESH,
            )
            d.start()
            fwd.append(d)

        for k in range(C):
            rs[k].wait_send()
            ag_x[k].wait_send()
            ag_y[k].wait()
            fwd[k].wait()

    return pl.pallas_call(
        body,
        out_shape=jax.ShapeDtypeStruct((m, 2 * n), jnp.bfloat16),
        in_specs=[pl.BlockSpec(memory_space=pltpu.VMEM)],
        out_specs=pl.BlockSpec(memory_space=pltpu.VMEM),
        scratch_shapes=[
            pltpu.VMEM((half, n), jnp.bfloat16),
            pltpu.VMEM((half, n), jnp.bfloat16),
            pltpu.SemaphoreType.DMA((C,)),
            pltpu.SemaphoreType.DMA((C,)),
            pltpu.SemaphoreType.DMA((C,)),
            pltpu.SemaphoreType.DMA((C,)),
            pltpu.SemaphoreType.DMA((C,)),
            pltpu.SemaphoreType.DMA((C,)),
            pltpu.SemaphoreType.DMA((C,)),
            pltpu.SemaphoreType.DMA((C,)),
        ],
        compiler_params=pltpu.CompilerParams(collective_id=0),
    )(x)
